# Initial kernel scaffold; baseline (speedup 1.0000x reference)
import jax
import jax.numpy as jnp
from jax.experimental import pallas as pl


def kernel(node_feats, edge_feats, edge_index, W_node, g_node, b_node, W_edge, g_edge, b_edge, W_out, g_out, b_out):
    raise NotImplementedError("write your pallas kernel here")


# speedup vs baseline: 54.5622x; 54.5622x over previous
import functools

import jax
import jax.numpy as jnp
from jax import lax
from jax.experimental import pallas as pl
from jax.experimental.pallas import tpu as pltpu
from jax.experimental.pallas import tpu_sc as plsc

N_NODES = 10000
N_EDGES = 320000
NODE_IN = 128
EDGE_IN = 16
HID = 128
OUT_FEATS = 32

_LN_EPS = 1e-5
_INV_SQRT2 = 0.7071067811865476


def _gelu(x):
    return 0.5 * x * (1.0 + lax.erf(x * _INV_SQRT2))


def _proj_node_body(x_ref, w_ref, g_ref, b_ref, o_ref):
    y = jnp.dot(x_ref[...], w_ref[...], preferred_element_type=jnp.float32)
    y = _gelu(y)
    mu = jnp.mean(y, axis=-1, keepdims=True)
    var = jnp.mean((y - mu) ** 2, axis=-1, keepdims=True)
    o_ref[...] = (y - mu) * lax.rsqrt(var + _LN_EPS) * g_ref[...] + b_ref[...]


def _proj_edge_body(x_ref, w_ref, g_ref, b_ref, o_ref):
    y = jnp.dot(x_ref[...], w_ref[...], preferred_element_type=jnp.float32)
    mu = jnp.mean(y, axis=-1, keepdims=True)
    var = jnp.mean((y - mu) ** 2, axis=-1, keepdims=True)
    o_ref[...] = jnp.exp((y - mu) * lax.rsqrt(var + _LN_EPS) * g_ref[...] + b_ref[...])


def _proj_out_body(h_ref, w_ref, g_ref, b_ref, o_ref):
    h = h_ref[0] + h_ref[1]
    y = jnp.dot(h, w_ref[...], preferred_element_type=jnp.float32)
    y = _gelu(y)
    mu = jnp.mean(y, axis=-1, keepdims=True)
    var = jnp.mean((y - mu) ** 2, axis=-1, keepdims=True)
    o_ref[...] = (y - mu) * lax.rsqrt(var + _LN_EPS) * g_ref[...] + b_ref[...]


_NODE_BLK = 1000
_EDGE_BLK = 8000


def _proj_node(x, w, g, b):
    grid = (N_NODES // _NODE_BLK,)
    return pl.pallas_call(
        _proj_node_body,
        grid=grid,
        in_specs=[
            pl.BlockSpec((_NODE_BLK, NODE_IN), lambda i: (i, 0)),
            pl.BlockSpec((NODE_IN, HID), lambda i: (0, 0)),
            pl.BlockSpec((1, HID), lambda i: (0, 0)),
            pl.BlockSpec((1, HID), lambda i: (0, 0)),
        ],
        out_specs=pl.BlockSpec((_NODE_BLK, HID), lambda i: (i, 0)),
        out_shape=jax.ShapeDtypeStruct((N_NODES, HID), jnp.float32),
    )(x, w, g, b)


def _proj_edge(x, w, g, b):
    grid = (N_EDGES // _EDGE_BLK,)
    return pl.pallas_call(
        _proj_edge_body,
        grid=grid,
        in_specs=[
            pl.BlockSpec((_EDGE_BLK, EDGE_IN), lambda i: (i, 0)),
            pl.BlockSpec((EDGE_IN, HID), lambda i: (0, 0)),
            pl.BlockSpec((1, HID), lambda i: (0, 0)),
            pl.BlockSpec((1, HID), lambda i: (0, 0)),
        ],
        out_specs=pl.BlockSpec((_EDGE_BLK, HID), lambda i: (i, 0)),
        out_shape=jax.ShapeDtypeStruct((N_EDGES, HID), jnp.float32),
    )(x, w, g, b)


def _proj_out(partials, w, g, b):
    grid = (N_NODES // _NODE_BLK,)
    return pl.pallas_call(
        _proj_out_body,
        grid=grid,
        in_specs=[
            pl.BlockSpec((2, _NODE_BLK, HID), lambda i: (0, i, 0)),
            pl.BlockSpec((HID, OUT_FEATS), lambda i: (0, 0)),
            pl.BlockSpec((1, OUT_FEATS), lambda i: (0, 0)),
            pl.BlockSpec((1, OUT_FEATS), lambda i: (0, 0)),
        ],
        out_specs=pl.BlockSpec((_NODE_BLK, OUT_FEATS), lambda i: (i, 0)),
        out_shape=jax.ShapeDtypeStruct((N_NODES, OUT_FEATS), jnp.float32),
    )(partials, w, g, b)


_NC = 2
_NS = 16
_NW = _NC * _NS
_C = 128
_CHUNKS = N_EDGES // _C
_CHUNKS_BASE = _CHUNKS // _NW
_CHUNKS_REM = _CHUNKS % _NW
_ROWS_PER_TILE = 632
_PAD_NODES = _ROWS_PER_TILE * _NS
_ZERO_CHUNKS = (128, 128, 128, 128, 120)
_LANES = HID // 16


def _sc_body(hv_hbm, he_hbm, src_hbm, dst_hbm, out_hbm,
             src_v, dst_v, g_v, e_v, sem, h_sh):
    c = lax.axis_index("c")
    s = lax.axis_index("s")
    w = s * _NC + c

    zero = jnp.zeros((16,), jnp.float32)

    def _zero_row(r, carry):
        for j in range(_LANES):
            g_v[r, pl.ds(j * 16, 16)] = zero
        return carry

    lax.fori_loop(0, _C, _zero_row, 0)
    off = 0
    for n in _ZERO_CHUNKS:
        pltpu.sync_copy(g_v.at[pl.ds(0, n)],
                        h_sh.at[pl.ds(s * _ROWS_PER_TILE + off, n)])
        off += n
    plsc.subcore_barrier()

    def _chunk(i, carry):
        base = (w + i * _NW) * _C
        pltpu.sync_copy(src_hbm.at[pl.ds(base, _C)], src_v.at[0])
        pltpu.sync_copy(dst_hbm.at[pl.ds(base, _C)], dst_v.at[0])
        gather = pltpu.async_copy(hv_hbm.at[src_v.at[0]], g_v, sem)
        pltpu.sync_copy(he_hbm.at[pl.ds(base, _C)], e_v)
        gather.wait()

        def _mul_row(r, cc):
            for j in range(_LANES):
                sl = pl.ds(j * 16, 16)
                g_v[r, sl] = g_v[r, sl] * e_v[r, sl]
            return cc

        lax.fori_loop(0, _C, _mul_row, 0)
        pltpu.sync_copy(g_v, h_sh.at[dst_v.at[0]], add=True)
        return carry

    n_chunks = _CHUNKS_BASE + jnp.where(w < _CHUNKS_REM, 1, 0)
    lax.fori_loop(0, n_chunks, _chunk, 0)

    plsc.subcore_barrier()
    pltpu.sync_copy(h_sh.at[pl.ds(s * _ROWS_PER_TILE, _ROWS_PER_TILE)],
                    out_hbm.at[c, pl.ds(s * _ROWS_PER_TILE, _ROWS_PER_TILE)])


_sc_gather_mul_scatter = functools.partial(
    pl.kernel,
    out_type=jax.ShapeDtypeStruct((_NC, _PAD_NODES, HID), jnp.float32),
    mesh=plsc.VectorSubcoreMesh(core_axis_name="c", subcore_axis_name="s",
                                num_cores=_NC, num_subcores=_NS),
    scratch_types=[
        pltpu.VMEM((1, _C), jnp.int32),
        pltpu.VMEM((1, _C), jnp.int32),
        pltpu.VMEM((_C, HID), jnp.float32),
        pltpu.VMEM((_C, HID), jnp.float32),
        pltpu.SemaphoreType.DMA,
        pltpu.VMEM_SHARED((_PAD_NODES, HID), jnp.float32),
    ],
)(_sc_body)


def kernel(node_feats, edge_feats, edge_index, W_node, g_node, b_node,
           W_edge, g_edge, b_edge, W_out, g_out, b_out):
    hv = _proj_node(node_feats, W_node,
                    g_node.reshape(1, -1), b_node.reshape(1, -1))
    he = _proj_edge(edge_feats, W_edge,
                    g_edge.reshape(1, -1), b_edge.reshape(1, -1))
    src = edge_index[0].astype(jnp.int32)
    dst = edge_index[1].astype(jnp.int32)
    partials = _sc_gather_mul_scatter(hv, he, src, dst)
    return _proj_out(partials, W_out,
                     g_out.reshape(1, -1), b_out.reshape(1, -1))
